# baseline (speedup 1.0000x reference)
import jax
import jax.numpy as jnp
from jax import lax
from jax.experimental import pallas as pl
from jax.experimental.pallas import tpu as pltpu

VOCAB_SIZE = 100000
EMB_DIM = 128
CH = 2000
NCH = VOCAB_SIZE // CH
NB = 4


def _tc_body(idx_ref, e_ref, w_hbm, b_ref, out_ref, wbuf, sem):
    e = e_ref[0]

    for k in range(NB):
        pltpu.make_async_copy(
            w_hbm.at[pl.ds(k * CH, CH)], wbuf.at[k], sem.at[k]
        ).start()

    def outer(o, _):
        for k in range(NB):
            c = o * NB + k
            pltpu.make_async_copy(
                w_hbm.at[pl.ds(c * CH, CH)], wbuf.at[k], sem.at[k]
            ).wait()
            logits = lax.dot_general(
                e, wbuf[k], (((1,), (1,)), ((), ())),
                preferred_element_type=jnp.float32,
            )
            out_ref[pl.ds(c, 1), :] = logits

            @pl.when(c + NB < NCH)
            def _():
                pltpu.make_async_copy(
                    w_hbm.at[pl.ds((c + NB) * CH, CH)], wbuf.at[k], sem.at[k]
                ).start()

        return 0

    lax.fori_loop(0, NCH // NB, outer, 0)

    for c in range((NCH // NB) * NB, NCH):
        k = c % NB
        pltpu.make_async_copy(
            w_hbm.at[pl.ds(c * CH, CH)], wbuf.at[k], sem.at[k]
        ).wait()
        logits = lax.dot_general(
            e, wbuf[k], (((1,), (1,)), ((), ())),
            preferred_element_type=jnp.float32,
        )
        out_ref[pl.ds(c, 1), :] = logits

    x = out_ref[...] + b_ref[...]
    m = jnp.max(x)
    lse = m + jnp.log(jnp.sum(jnp.exp(x - m)))
    out_ref[...] = x - lse


def _tc_linear_logsoftmax(idx, emb_table, W, b):
    grid_spec = pltpu.PrefetchScalarGridSpec(
        num_scalar_prefetch=1,
        grid=(1,),
        in_specs=[
            pl.BlockSpec((1, 1, EMB_DIM), lambda i, idx_ref: (idx_ref[0], 0, 0)),
            pl.BlockSpec(memory_space=pltpu.MemorySpace.HBM),
            pl.BlockSpec((NCH, CH), lambda i, idx_ref: (0, 0)),
        ],
        out_specs=pl.BlockSpec((NCH, CH), lambda i, idx_ref: (0, 0)),
        scratch_shapes=[
            pltpu.VMEM((NB, CH, EMB_DIM), jnp.float32),
            pltpu.SemaphoreType.DMA((NB,)),
        ],
    )
    return pl.pallas_call(
        _tc_body,
        grid_spec=grid_spec,
        out_shape=jax.ShapeDtypeStruct((NCH, CH), jnp.float32),
    )(idx, emb_table.reshape(VOCAB_SIZE, 1, EMB_DIM), W, b.reshape(NCH, CH))


def kernel(inputs, emb_table, W, b):
    idx = inputs.astype(jnp.int32)
    out = _tc_linear_logsoftmax(idx, emb_table, W, b)
    return out.reshape(1, VOCAB_SIZE)


# speedup vs baseline: 1.0885x; 1.0885x over previous
import jax
import jax.numpy as jnp
from jax import lax
from jax.experimental import pallas as pl
from jax.experimental.pallas import tpu as pltpu

VOCAB_SIZE = 100000
EMB_DIM = 128
BLK = 10000
NBLK = VOCAB_SIZE // BLK


def _tc_body(idx_ref, e_ref, w_ref, b_ref, out_ref):
    i = pl.program_id(0)

    e = e_ref[0]
    logits = lax.dot_general(
        e, w_ref[...], (((1,), (1,)), ((), ())), preferred_element_type=jnp.float32
    )
    out_ref[pl.ds(i, 1), :] = logits

    @pl.when(i == NBLK - 1)
    def _():
        x = out_ref[...] + b_ref[...]
        m = jnp.max(x)
        lse = m + jnp.log(jnp.sum(jnp.exp(x - m)))
        out_ref[...] = x - lse


def _tc_linear_logsoftmax(idx, emb_table, W, b):
    grid_spec = pltpu.PrefetchScalarGridSpec(
        num_scalar_prefetch=1,
        grid=(NBLK,),
        in_specs=[
            pl.BlockSpec((1, 1, EMB_DIM), lambda i, idx_ref: (idx_ref[0], 0, 0)),
            pl.BlockSpec((BLK, EMB_DIM), lambda i, idx_ref: (i, 0)),
            pl.BlockSpec((NBLK, BLK), lambda i, idx_ref: (0, 0)),
        ],
        out_specs=pl.BlockSpec((NBLK, BLK), lambda i, idx_ref: (0, 0)),
    )
    return pl.pallas_call(
        _tc_body,
        grid_spec=grid_spec,
        out_shape=jax.ShapeDtypeStruct((NBLK, BLK), jnp.float32),
    )(idx, emb_table.reshape(VOCAB_SIZE, 1, EMB_DIM), W, b.reshape(NBLK, BLK))


def kernel(inputs, emb_table, W, b):
    idx = inputs.astype(jnp.int32)
    out = _tc_linear_logsoftmax(idx, emb_table, W, b)
    return out.reshape(1, VOCAB_SIZE)
